# baseline (speedup 1.0000x reference)
import functools

import jax
import jax.numpy as jnp
from jax import lax
from jax.experimental import pallas as pl
from jax.experimental.pallas import tpu as pltpu
from jax.experimental.pallas import tpu_sc as plsc

N_USERS = 5000
N_ITEMS = 5000
N = N_USERS + N_ITEMS
NPAD = 10240
D = 128
E = 320000
NC = 2
NS = 16
NW = NC * NS
EPT = E // NW
K = 16
NCHUNK = EPT // K
NB = 12
KD = 80
RPS = NPAD // NS

_F32 = jnp.float32

_mesh = plsc.VectorSubcoreMesh(core_axis_name="c", subcore_axis_name="s")


def _deg_body(colc_hbm, colr_hbm, degc_hbm, degr_hbm,
              idx_v, ones_v, zdeg_v, acc_c, acc_r):
  c = lax.axis_index("c")
  s = lax.axis_index("s")
  wid = s * NC + c
  one16 = jnp.ones((16,), _F32)
  zero16 = jnp.zeros((16,), _F32)
  for j in range(KD // 16):
    ones_v[pl.ds(j * 16, 16)] = one16

  def zfill(i, carry):
    zdeg_v[pl.ds(i * 16, 16)] = zero16
    return carry
  lax.fori_loop(0, RPS // 16, zfill, 0)

  pltpu.sync_copy(zdeg_v, acc_c.at[pl.ds(s * RPS, RPS)])
  pltpu.sync_copy(zdeg_v, acc_r.at[pl.ds(s * RPS, RPS)])
  plsc.subcore_barrier()

  for col_hbm, acc in ((colc_hbm, acc_c), (colr_hbm, acc_r)):
    pltpu.sync_copy(col_hbm.at[wid], idx_v)

    def chunk(g, carry):
      pltpu.sync_copy(ones_v, acc.at[idx_v.at[g]], add=True)
      return carry
    lax.fori_loop(0, EPT // KD, chunk, 0)

  plsc.subcore_barrier()
  pltpu.sync_copy(acc_c.at[pl.ds(s * RPS, RPS)],
                  degc_hbm.at[c, pl.ds(s * RPS, RPS)])
  pltpu.sync_copy(acc_r.at[pl.ds(s * RPS, RPS)],
                  degr_hbm.at[c, pl.ds(s * RPS, RPS)])


_deg_kernel = functools.partial(
    pl.kernel,
    out_type=(jax.ShapeDtypeStruct((NC, NPAD), _F32),
              jax.ShapeDtypeStruct((NC, NPAD), _F32)),
    mesh=_mesh,
    scratch_types=[
        pltpu.VMEM((EPT // KD, KD), jnp.int32),
        pltpu.VMEM((KD,), _F32),
        pltpu.VMEM((RPS,), _F32),
        pltpu.VMEM_SHARED((NPAD,), _F32),
        pltpu.VMEM_SHARED((NPAD,), _F32),
    ],
)(_deg_body)


ZF = max(d for d in range(1, K + 1) if RPS % d == 0)


def _apply_body(y_hbm, row_hbm, col_hbm, part_hbm,
                rows_v, cols_v, acc, *bufs):
  dbufs = bufs[:NB]
  gsems = bufs[NB:2 * NB]
  ssems = bufs[2 * NB:3 * NB]
  c = lax.axis_index("c")
  s = lax.axis_index("s")
  wid = s * NC + c
  zero16 = jnp.zeros((16,), _F32)

  def zrow(i, carry):
    for j in range(D // 16):
      dbufs[0][i, pl.ds(j * 16, 16)] = zero16
    return carry
  lax.fori_loop(0, ZF, zrow, 0)

  def zacc(k, carry):
    pltpu.sync_copy(dbufs[0].at[pl.ds(0, ZF)],
                    acc.at[pl.ds(s * RPS + k * ZF, ZF)])
    return carry
  lax.fori_loop(0, RPS // ZF, zacc, 0)

  pltpu.sync_copy(row_hbm.at[wid], rows_v)
  pltpu.sync_copy(col_hbm.at[wid], cols_v)
  plsc.subcore_barrier()

  def idx(iv, g):
    return iv.at[pl.ds(g * K, K)]

  def gather(g, b):
    return pltpu.make_async_copy(y_hbm.at[idx(rows_v, g)], dbufs[b], gsems[b])

  def scatter(g, b):
    return pltpu.async_copy(dbufs[b], acc.at[idx(cols_v, g)], ssems[b],
                            add=True)

  def scatter_wait(g, b):
    pltpu.make_async_copy(dbufs[b], acc.at[idx(cols_v, g)], ssems[b]).wait()

  for b in range(NB - 1):
    pltpu.async_copy(y_hbm.at[idx(rows_v, b)], dbufs[b], gsems[b])

  def ring(t, carry):
    for b in range(NB):
      g = NB * t + b

      @pl.when(g < NCHUNK)
      def _():
        gather(g, b).wait()
        scatter(g, b)
        gi = g + NB - 1
        b2 = (b + NB - 1) % NB

        @pl.when(gi - NB >= 0)
        def _():
          scatter_wait(gi - NB, b2)

        @pl.when(gi < NCHUNK)
        def _():
          pltpu.async_copy(y_hbm.at[idx(rows_v, gi)], dbufs[b2], gsems[b2])
    return carry
  lax.fori_loop(0, (NCHUNK + NB - 1) // NB, ring, 0)

  scatter_wait(NCHUNK - 1, (NCHUNK - 1) % NB)

  plsc.subcore_barrier()
  pltpu.sync_copy(acc.at[pl.ds(s * RPS, RPS)],
                  part_hbm.at[c, pl.ds(s * RPS, RPS)])


_apply_kernel = functools.partial(
    pl.kernel,
    out_type=jax.ShapeDtypeStruct((NC, NPAD, D), _F32),
    mesh=_mesh,
    scratch_types=[
        pltpu.VMEM((EPT,), jnp.int32),
        pltpu.VMEM((EPT,), jnp.int32),
        pltpu.VMEM_SHARED((NPAD, D), _F32),
    ] + [pltpu.VMEM((K, D), _F32) for _ in range(NB)]
      + [pltpu.SemaphoreType.DMA for _ in range(2 * NB)],
)(_apply_body)


def _dis_of(deg):
  return jnp.where(deg > 0, 1.0 / jnp.sqrt(jnp.where(deg > 0, deg, 1.0)), 0.0)


def _prep_body(dcp, drp, u, it, dis_c_o, dis_r_o, y0_o):
  dis_c = _dis_of(dcp[0] + dcp[1])
  dis_r = _dis_of(drp[0] + drp[1])
  dis_c_o[...] = dis_c
  dis_r_o[...] = dis_r
  x0 = jnp.concatenate(
      [u[...], it[...], jnp.zeros((NPAD - N, D), _F32)], axis=0)
  y0_o[...] = x0 * dis_c[:, None]


def _prep_tc(dcp, drp, u, it):
  return pl.pallas_call(
      _prep_body,
      out_shape=(jax.ShapeDtypeStruct((NPAD,), _F32),
                 jax.ShapeDtypeStruct((NPAD,), _F32),
                 jax.ShapeDtypeStruct((NPAD, D), _F32)),
  )(dcp, drp, u, it)


def _mid_body(tp, dis, h_o, ynext_o):
  h = (tp[0] + tp[1]) * dis[...][:, None]
  h_o[...] = h
  ynext_o[...] = h * dis[...][:, None]


def _mid_tc(tp, dis):
  return pl.pallas_call(
      _mid_body,
      out_shape=(jax.ShapeDtypeStruct((NPAD, D), _F32),
                 jax.ShapeDtypeStruct((NPAD, D), _F32)),
  )(tp, dis)


def _cartend_body(tp, dis_c, u, it, h1, wu, wi, dis_r,
                  cart_o, xr_o, y0r_o):
  h2 = (tp[0] + tp[1]) * dis_c[...][:, None]
  x0 = jnp.concatenate(
      [u[...], it[...], jnp.zeros((NPAD - N, D), _F32)], axis=0)
  cart = (x0 + h1[...] + h2) * (1.0 / 3.0)
  cart_o[...] = cart
  up = jnp.dot(cart[:N_USERS], wu[...].T, preferred_element_type=_F32)
  ip = jnp.dot(cart[N_USERS:N], wi[...].T, preferred_element_type=_F32)
  xr = jnp.concatenate([up, ip, jnp.zeros((NPAD - N, D), _F32)], axis=0)
  xr_o[...] = xr
  y0r_o[...] = xr * dis_r[...][:, None]


def _cartend_tc(tp, dis_c, u, it, h1, wu, wi, dis_r):
  return pl.pallas_call(
      _cartend_body,
      out_shape=(jax.ShapeDtypeStruct((NPAD, D), _F32),
                 jax.ShapeDtypeStruct((NPAD, D), _F32),
                 jax.ShapeDtypeStruct((NPAD, D), _F32)),
  )(tp, dis_c, u, it, h1, wu, wi, dis_r)


def _final_body(tp, dis_r, xr, h1r, cart, u_o, i_o):
  h2r = (tp[0] + tp[1]) * dis_r[...][:, None]
  rent = (xr[...] + h1r[...] + h2r) * (1.0 / 3.0)
  tot = cart[...] + rent
  u_o[...] = tot[:N_USERS]
  i_o[...] = tot[N_USERS:N]


def _final_tc(tp, dis_r, xr, h1r, cart):
  return pl.pallas_call(
      _final_body,
      out_shape=(jax.ShapeDtypeStruct((N_USERS, D), _F32),
                 jax.ShapeDtypeStruct((N_ITEMS, D), _F32)),
  )(tp, dis_r, xr, h1r, cart)


def kernel(edge_index_cart, edge_index_rent, user_emb, item_emb,
           W_user, W_item):
  ec = edge_index_cart.astype(jnp.int32)
  er = edge_index_rent.astype(jnp.int32)
  rc = ec[0].reshape(NW, EPT)
  cc = ec[1].reshape(NW, EPT)
  rr = er[0].reshape(NW, EPT)
  cr = er[1].reshape(NW, EPT)

  degc_p, degr_p = _deg_kernel(ec[1].reshape(NW, EPT // KD, KD),
                               er[1].reshape(NW, EPT // KD, KD))
  dis_c, dis_r, y0 = _prep_tc(degc_p, degr_p, user_emb, item_emb)

  t1p = _apply_kernel(y0, rc, cc)
  h1, y1 = _mid_tc(t1p, dis_c)
  t2p = _apply_kernel(y1, rc, cc)
  cart, xr, y0r = _cartend_tc(t2p, dis_c, user_emb, item_emb, h1,
                              W_user, W_item, dis_r)

  t1rp = _apply_kernel(y0r, rr, cr)
  h1r, y1r = _mid_tc(t1rp, dis_r)
  t2rp = _apply_kernel(y1r, rr, cr)
  return _final_tc(t2rp, dis_r, xr, h1r, cart)


# speedup vs baseline: 1.1301x; 1.1301x over previous
import functools

import jax
import jax.numpy as jnp
from jax import lax
from jax.experimental import pallas as pl
from jax.experimental.pallas import tpu as pltpu
from jax.experimental.pallas import tpu_sc as plsc

N_USERS = 5000
N_ITEMS = 5000
N = N_USERS + N_ITEMS
NPAD = 10240
D = 128
E = 320000
NC = 2
NS = 16
NW = NC * NS
EPT = E // NW
K = 40
NCHUNK = EPT // K
NB = 8
SEG = 5
SEGE = EPT // SEG
SCH = SEGE // K
KD = 80
RPS = NPAD // NS

_F32 = jnp.float32

_mesh = plsc.VectorSubcoreMesh(core_axis_name="c", subcore_axis_name="s")


def _deg_body(colc_hbm, colr_hbm, degc_hbm, degr_hbm,
              idx_v, ones_v, zdeg_v, acc_c, acc_r):
  c = lax.axis_index("c")
  s = lax.axis_index("s")
  wid = s * NC + c
  one16 = jnp.ones((16,), _F32)
  zero16 = jnp.zeros((16,), _F32)
  for j in range(KD // 16):
    ones_v[pl.ds(j * 16, 16)] = one16

  def zfill(i, carry):
    zdeg_v[pl.ds(i * 16, 16)] = zero16
    return carry
  lax.fori_loop(0, RPS // 16, zfill, 0)

  pltpu.sync_copy(zdeg_v, acc_c.at[pl.ds(s * RPS, RPS)])
  pltpu.sync_copy(zdeg_v, acc_r.at[pl.ds(s * RPS, RPS)])
  plsc.subcore_barrier()

  for col_hbm, acc in ((colc_hbm, acc_c), (colr_hbm, acc_r)):
    pltpu.sync_copy(col_hbm.at[wid], idx_v)

    def chunk(g, carry):
      pltpu.sync_copy(ones_v, acc.at[idx_v.at[g]], add=True)
      return carry
    lax.fori_loop(0, EPT // KD, chunk, 0)

  plsc.subcore_barrier()
  pltpu.sync_copy(acc_c.at[pl.ds(s * RPS, RPS)],
                  degc_hbm.at[c, pl.ds(s * RPS, RPS)])
  pltpu.sync_copy(acc_r.at[pl.ds(s * RPS, RPS)],
                  degr_hbm.at[c, pl.ds(s * RPS, RPS)])


_deg_kernel = functools.partial(
    pl.kernel,
    out_type=(jax.ShapeDtypeStruct((NC, NPAD), _F32),
              jax.ShapeDtypeStruct((NC, NPAD), _F32)),
    mesh=_mesh,
    scratch_types=[
        pltpu.VMEM((EPT // KD, KD), jnp.int32),
        pltpu.VMEM((KD,), _F32),
        pltpu.VMEM((RPS,), _F32),
        pltpu.VMEM_SHARED((NPAD,), _F32),
        pltpu.VMEM_SHARED((NPAD,), _F32),
    ],
)(_deg_body)


ZF = max(d for d in range(1, K + 1) if RPS % d == 0)


def _apply_body(y_hbm, row_hbm, col_hbm, part_hbm,
                rows_va, rows_vb, cols_va, cols_vb, acc, prsem, pcsem, *bufs):
  dbufs = bufs[:NB]
  gsems = bufs[NB:2 * NB]
  ssems = bufs[2 * NB:3 * NB]
  c = lax.axis_index("c")
  s = lax.axis_index("s")
  wid = s * NC + c
  zero16 = jnp.zeros((16,), _F32)

  def zrow(i, carry):
    for j in range(D // 16):
      dbufs[0][i, pl.ds(j * 16, 16)] = zero16
    return carry
  lax.fori_loop(0, ZF, zrow, 0)

  def zacc(k, carry):
    pltpu.sync_copy(dbufs[0].at[pl.ds(0, ZF)],
                    acc.at[pl.ds(s * RPS + k * ZF, ZF)])
    return carry
  lax.fori_loop(0, RPS // ZF, zacc, 0)

  pltpu.sync_copy(row_hbm.at[wid * SEG], rows_va)
  pltpu.sync_copy(col_hbm.at[wid * SEG], cols_va)
  pltpu.async_copy(row_hbm.at[wid * SEG + 1], rows_vb, prsem)
  pltpu.async_copy(col_hbm.at[wid * SEG + 1], cols_vb, pcsem)
  plsc.subcore_barrier()

  def view(iv, g):
    return iv.at[pl.ds(lax.rem(g, SCH) * K, K)]

  def issue_gather(g, b):
    par = lax.rem(lax.div(g, SCH), 2)

    @pl.when(par == 0)
    def _():
      pltpu.async_copy(y_hbm.at[view(rows_va, g)], dbufs[b], gsems[b])

    @pl.when(par == 1)
    def _():
      pltpu.async_copy(y_hbm.at[view(rows_vb, g)], dbufs[b], gsems[b])

  def gather_wait(g, b):
    pltpu.make_async_copy(y_hbm.at[view(rows_va, g)], dbufs[b],
                          gsems[b]).wait()

  def issue_scatter(g, b):
    par = lax.rem(lax.div(g, SCH), 2)

    @pl.when(par == 0)
    def _():
      pltpu.async_copy(dbufs[b], acc.at[view(cols_va, g)], ssems[b],
                       add=True)

    @pl.when(par == 1)
    def _():
      pltpu.async_copy(dbufs[b], acc.at[view(cols_vb, g)], ssems[b],
                       add=True)

  def scatter_wait(g, b):
    pltpu.make_async_copy(dbufs[b], acc.at[view(cols_va, g)],
                          ssems[b]).wait()

  for b in range(NB - 1):
    issue_gather(b, b)

  def ring(t, carry):
    for b in range(NB):
      g = NB * t + b

      @pl.when(g < NCHUNK)
      def _():
        sg = lax.div(g, SCH)
        lc = lax.rem(g, SCH)
        gather_wait(g, b)
        issue_scatter(g, b)
        gi = g + NB - 1
        b2 = (b + NB - 1) % NB

        @pl.when(gi - NB >= 0)
        def _():
          scatter_wait(gi - NB, b2)

        @pl.when(jnp.logical_and(lc == 1,
                                 jnp.logical_and(sg >= 1, sg <= SEG - 2)))
        def _():
          tpar = lax.rem(sg + 1, 2)

          @pl.when(tpar == 0)
          def _():
            pltpu.async_copy(row_hbm.at[wid * SEG + sg + 1], rows_va, prsem)
            pltpu.async_copy(col_hbm.at[wid * SEG + sg + 1], cols_va, pcsem)

          @pl.when(tpar == 1)
          def _():
            pltpu.async_copy(row_hbm.at[wid * SEG + sg + 1], rows_vb, prsem)
            pltpu.async_copy(col_hbm.at[wid * SEG + sg + 1], cols_vb, pcsem)

        @pl.when(jnp.logical_and(lc == SCH - NB + 1, sg <= SEG - 2))
        def _():
          pltpu.make_async_copy(row_hbm.at[wid * SEG], rows_va, prsem).wait()
          pltpu.make_async_copy(col_hbm.at[wid * SEG], cols_va, pcsem).wait()

        @pl.when(gi < NCHUNK)
        def _():
          issue_gather(gi, b2)
    return carry
  lax.fori_loop(0, (NCHUNK + NB - 1) // NB, ring, 0)

  scatter_wait(NCHUNK - 1, (NCHUNK - 1) % NB)

  plsc.subcore_barrier()
  pltpu.sync_copy(acc.at[pl.ds(s * RPS, RPS)],
                  part_hbm.at[c, pl.ds(s * RPS, RPS)])


_apply_kernel = functools.partial(
    pl.kernel,
    out_type=jax.ShapeDtypeStruct((NC, NPAD, D), _F32),
    mesh=_mesh,
    scratch_types=[
        pltpu.VMEM((SEGE,), jnp.int32),
        pltpu.VMEM((SEGE,), jnp.int32),
        pltpu.VMEM((SEGE,), jnp.int32),
        pltpu.VMEM((SEGE,), jnp.int32),
        pltpu.VMEM_SHARED((NPAD, D), _F32),
        pltpu.SemaphoreType.DMA,
        pltpu.SemaphoreType.DMA,
    ] + [pltpu.VMEM((K, D), _F32) for _ in range(NB)]
      + [pltpu.SemaphoreType.DMA for _ in range(2 * NB)],
)(_apply_body)


def _dis_of(deg):
  return jnp.where(deg > 0, 1.0 / jnp.sqrt(jnp.where(deg > 0, deg, 1.0)), 0.0)


def _prep_body(dcp, drp, u, it, dis_c_o, dis_r_o, y0_o):
  dis_c = _dis_of(dcp[0] + dcp[1])
  dis_r = _dis_of(drp[0] + drp[1])
  dis_c_o[...] = dis_c
  dis_r_o[...] = dis_r
  x0 = jnp.concatenate(
      [u[...], it[...], jnp.zeros((NPAD - N, D), _F32)], axis=0)
  y0_o[...] = x0 * dis_c[:, None]


def _prep_tc(dcp, drp, u, it):
  return pl.pallas_call(
      _prep_body,
      out_shape=(jax.ShapeDtypeStruct((NPAD,), _F32),
                 jax.ShapeDtypeStruct((NPAD,), _F32),
                 jax.ShapeDtypeStruct((NPAD, D), _F32)),
  )(dcp, drp, u, it)


def _mid_body(tp, dis, h_o, ynext_o):
  h = (tp[0] + tp[1]) * dis[...][:, None]
  h_o[...] = h
  ynext_o[...] = h * dis[...][:, None]


def _mid_tc(tp, dis):
  return pl.pallas_call(
      _mid_body,
      out_shape=(jax.ShapeDtypeStruct((NPAD, D), _F32),
                 jax.ShapeDtypeStruct((NPAD, D), _F32)),
  )(tp, dis)


def _cartend_body(tp, dis_c, u, it, h1, wu, wi, dis_r,
                  cart_o, xr_o, y0r_o):
  h2 = (tp[0] + tp[1]) * dis_c[...][:, None]
  x0 = jnp.concatenate(
      [u[...], it[...], jnp.zeros((NPAD - N, D), _F32)], axis=0)
  cart = (x0 + h1[...] + h2) * (1.0 / 3.0)
  cart_o[...] = cart
  up = jnp.dot(cart[:N_USERS], wu[...].T, preferred_element_type=_F32)
  ip = jnp.dot(cart[N_USERS:N], wi[...].T, preferred_element_type=_F32)
  xr = jnp.concatenate([up, ip, jnp.zeros((NPAD - N, D), _F32)], axis=0)
  xr_o[...] = xr
  y0r_o[...] = xr * dis_r[...][:, None]


def _cartend_tc(tp, dis_c, u, it, h1, wu, wi, dis_r):
  return pl.pallas_call(
      _cartend_body,
      out_shape=(jax.ShapeDtypeStruct((NPAD, D), _F32),
                 jax.ShapeDtypeStruct((NPAD, D), _F32),
                 jax.ShapeDtypeStruct((NPAD, D), _F32)),
  )(tp, dis_c, u, it, h1, wu, wi, dis_r)


def _final_body(tp, dis_r, xr, h1r, cart, u_o, i_o):
  h2r = (tp[0] + tp[1]) * dis_r[...][:, None]
  rent = (xr[...] + h1r[...] + h2r) * (1.0 / 3.0)
  tot = cart[...] + rent
  u_o[...] = tot[:N_USERS]
  i_o[...] = tot[N_USERS:N]


def _final_tc(tp, dis_r, xr, h1r, cart):
  return pl.pallas_call(
      _final_body,
      out_shape=(jax.ShapeDtypeStruct((N_USERS, D), _F32),
                 jax.ShapeDtypeStruct((N_ITEMS, D), _F32)),
  )(tp, dis_r, xr, h1r, cart)


def kernel(edge_index_cart, edge_index_rent, user_emb, item_emb,
           W_user, W_item):
  ec = edge_index_cart.astype(jnp.int32)
  er = edge_index_rent.astype(jnp.int32)
  rc = ec[0].reshape(NW * SEG, SEGE)
  cc = ec[1].reshape(NW * SEG, SEGE)
  rr = er[0].reshape(NW * SEG, SEGE)
  cr = er[1].reshape(NW * SEG, SEGE)

  degc_p, degr_p = _deg_kernel(ec[1].reshape(NW, EPT // KD, KD),
                               er[1].reshape(NW, EPT // KD, KD))
  dis_c, dis_r, y0 = _prep_tc(degc_p, degr_p, user_emb, item_emb)

  t1p = _apply_kernel(y0, rc, cc)
  h1, y1 = _mid_tc(t1p, dis_c)
  t2p = _apply_kernel(y1, rc, cc)
  cart, xr, y0r = _cartend_tc(t2p, dis_c, user_emb, item_emb, h1,
                              W_user, W_item, dis_r)

  t1rp = _apply_kernel(y0r, rr, cr)
  h1r, y1r = _mid_tc(t1rp, dis_r)
  t2rp = _apply_kernel(y1r, rr, cr)
  return _final_tc(t2rp, dis_r, xr, h1r, cart)
